# baseline (speedup 1.0000x reference)
import jax
import jax.numpy as jnp
from jax.experimental import pallas as pl

H = 50
W = 50
C = 512
PW = W + 2
M = H * PW
NH = 128


def _rpn_kernel(x_ref, w_ref, bsw_ref, wh_ref, bh_ref, out_ref):
    acc = jnp.zeros((M, C), dtype=jnp.float32)
    for ky in range(3):
        for kx in range(3):
            s = ky * PW + kx
            acc = acc + jnp.dot(
                x_ref[s:s + M, :], w_ref[ky * 3 + kx],
                preferred_element_type=jnp.float32)
    feats = jnp.maximum(acc + bsw_ref[:], 0.0).astype(jnp.bfloat16)
    out_ref[:, :] = (
        jnp.dot(feats, wh_ref[:], preferred_element_type=jnp.float32)
        + bh_ref[:])


def kernel(x, W_sw, b_sw, W_cls, b_cls, W_reg, b_reg):
    xhwc = jnp.transpose(x[0], (1, 2, 0))
    xpad = jnp.pad(xhwc, ((1, 1), (1, 1), (0, 0)))
    xflat = jnp.pad(xpad.reshape(PW * PW, C), ((0, 8), (0, 0)))
    xflat = xflat.astype(jnp.bfloat16)

    w9 = jnp.transpose(W_sw, (2, 3, 1, 0)).reshape(9, C, C).astype(jnp.bfloat16)
    bsw = b_sw.reshape(1, C)

    wh = jnp.concatenate(
        [W_reg.reshape(36, C), W_cls.reshape(18, C),
         jnp.zeros((NH - 54, C), jnp.float32)], axis=0)
    wh = jnp.transpose(wh, (1, 0)).astype(jnp.bfloat16)
    bh = jnp.concatenate(
        [b_reg, b_cls, jnp.zeros((NH - 54,), jnp.float32)]).reshape(1, NH)

    out = pl.pallas_call(
        _rpn_kernel,
        out_shape=jax.ShapeDtypeStruct((M, NH), jnp.float32),
    )(xflat, w9, bsw, wh, bh)

    o = out.reshape(H, PW, NH)[:, :W, :]
    reg = o[:, :, :36].reshape(1, H * W * 9, 4)
    cls = o[:, :, 36:54].reshape(1, H * W * 9, 2)
    return (reg, cls)
